# Initial kernel scaffold; baseline (speedup 1.0000x reference)
import jax
import jax.numpy as jnp
from jax.experimental import pallas as pl


def kernel(edge_index, node_features):
    raise NotImplementedError("write your pallas kernel here")


# speedup vs baseline: 17.7294x; 17.7294x over previous
import math
import functools

import jax
import jax.numpy as jnp
from jax import lax
from jax.experimental import pallas as pl
from jax.experimental.pallas import tpu as pltpu
from jax.experimental.pallas import tpu_sc as plsc

N_NODES = 10000
N_EDGES = 320000
N_ENDPOINTS = 2 * N_EDGES
NPAD = 10240
NC = 2
NS = 16
NW = NC * NS
L = 16
CHUNK = N_ENDPOINTS // NW


def _sc_hist_body(edges_hbm, out_hbm, idx_v, hist_v):
    c = lax.axis_index("c")
    s = lax.axis_index("s")
    wid = s * NC + c
    base = wid * CHUNK
    pltpu.sync_copy(edges_hbm.at[pl.ds(base, CHUNK)], idx_v)

    zeros = jnp.zeros((L,), jnp.float32)

    def zero_body(i, carry):
        hist_v[pl.ds(i * L, L)] = zeros
        return carry

    lax.fori_loop(0, NPAD // L, zero_body, 0)

    ones = jnp.ones((L,), jnp.float32)

    def add_body(i, carry):
        idx = idx_v[pl.ds(i * L, L)]
        plsc.addupdate_scatter(hist_v, [idx], ones)
        return carry

    lax.fori_loop(0, CHUNK // L, add_body, 0)

    pltpu.sync_copy(hist_v, out_hbm.at[wid])


_sc_hist = functools.partial(
    pl.kernel,
    mesh=plsc.VectorSubcoreMesh(core_axis_name="c", subcore_axis_name="s"),
    out_type=jax.ShapeDtypeStruct((NW, NPAD), jnp.float32),
    scratch_types=[
        pltpu.VMEM((CHUNK,), jnp.int32),
        pltpu.VMEM((NPAD,), jnp.float32),
    ],
    compiler_params=pltpu.CompilerParams(needs_layout_passes=False),
)(_sc_hist_body)


def _tc_stats_body(parts_ref, out_ref):
    x = parts_ref[...]
    deg = jnp.sum(x, axis=0, keepdims=True)
    col = lax.broadcasted_iota(jnp.int32, (1, NPAD), 1)
    valid = col < N_NODES

    total = jnp.sum(deg)
    max_deg = jnp.max(deg)
    mean = total / N_NODES
    centered = jnp.where(valid, deg - mean, 0.0)
    var = jnp.sum(centered * centered) / (N_NODES - 1)

    normalized_var = var / (mean + 1e-8)
    clustering = jnp.minimum(jnp.float32(1.0), normalized_var * 0.1)
    clustering = jnp.where(max_deg <= 1.0, jnp.float32(0.0), clustering)

    avg_degree = 2.0 * N_EDGES / N_NODES
    out_ref[0] = jnp.float32(math.log(N_NODES))
    out_ref[1] = jnp.float32(math.log(N_EDGES))
    out_ref[2] = jnp.float32(avg_degree)
    out_ref[3] = clustering
    out_ref[4] = jnp.float32(math.log(N_NODES) / math.log(max(2, avg_degree)))
    out_ref[5] = jnp.float32(2.0 * N_EDGES / (N_NODES * (N_NODES - 1)))


_tc_stats = pl.pallas_call(
    _tc_stats_body,
    out_shape=jax.ShapeDtypeStruct((6,), jnp.float32),
    out_specs=pl.BlockSpec(memory_space=pltpu.SMEM),
)


def kernel(edge_index, node_features):
    del node_features
    endpoints = edge_index.reshape(-1).astype(jnp.int32)
    parts = _sc_hist(endpoints)
    return _tc_stats(parts)
